# baseline (speedup 1.0000x reference)
import functools

import jax
import jax.numpy as jnp
from jax import lax
from jax.experimental import pallas as pl
from jax.experimental.pallas import tpu as pltpu
from jax.experimental.pallas import tpu_sc as plsc

_MEM = 100000
_DIM = 64
_B = 1024
_BLK = 2048
_CH = 128
_NCH = _BLK // _CH
_RG = 128
_NBLK = -(-_MEM // _BLK)
_THR = 0.8
_BIG = 0x3FFFFFFF
_EPS = 7.6e-6


def _norm_x(xt):
    n = jnp.sqrt(jnp.sum(xt * xt, axis=0, keepdims=True))
    return (xt / jnp.maximum(n, 1e-12)).astype(jnp.bfloat16)


def _norm_block(mt_ref, pid):
    col = lax.broadcasted_iota(jnp.int32, (1, _BLK), 1)
    valid = (col + pid * _BLK) < _MEM
    mv = jnp.where(valid, mt_ref[...], 0.0)
    nsq = jnp.sum(mv * mv, axis=0, keepdims=True)
    rnorm = 1.0 / jnp.maximum(jnp.sqrt(nsq), 1e-12)
    return mv, (mv * rnorm).astype(jnp.bfloat16)


def _max_body(xt_ref, mt_ref, maxv_ref, rowmaj_ref, xn_ref, run_ref):
    pid = pl.program_id(0)

    @pl.when(pid == 0)
    def _init():
        xn_ref[...] = _norm_x(xt_ref[...])
        run_ref[...] = jnp.full((_B, _CH), -jnp.inf, jnp.float32)

    mv, mn = _norm_block(mt_ref, pid)
    rowmaj_ref[...] = mv.T

    sim = lax.dot_general(xn_ref[...], mn, (((0,), (0,)), ((), ())),
                          preferred_element_type=jnp.float32)
    for r in range(_B // _RG):
        acc = run_ref[pl.ds(r * _RG, _RG), :]
        for k in range(_NCH):
            acc = jnp.maximum(acc, lax.slice(
                sim, (r * _RG, k * _CH), ((r + 1) * _RG, (k + 1) * _CH)))
        run_ref[pl.ds(r * _RG, _RG), :] = acc

    @pl.when(pid == _NBLK - 1)
    def _fin():
        maxv_ref[...] = jnp.max(run_ref[...], axis=1, keepdims=True)


@functools.lru_cache(maxsize=1)
def _max_call():
    return pl.pallas_call(
        _max_body,
        grid=(_NBLK,),
        in_specs=[
            pl.BlockSpec((_DIM, _B), lambda i: (0, 0)),
            pl.BlockSpec((_DIM, _BLK), lambda i: (0, i)),
        ],
        out_specs=[
            pl.BlockSpec((_B, 1), lambda i: (0, 0)),
            pl.BlockSpec((_BLK, _DIM), lambda i: (i, 0)),
        ],
        out_shape=[
            jax.ShapeDtypeStruct((_B, 1), jnp.float32),
            jax.ShapeDtypeStruct((_NBLK * _BLK, _DIM), jnp.float32),
        ],
        scratch_shapes=[
            pltpu.VMEM((_DIM, _B), jnp.bfloat16),
            pltpu.VMEM((_B, _CH), jnp.float32),
        ],
    )


def _idx_body(flag_ref, xt_ref, mt_ref, maxv_ref, maxi_ref, xn_ref, runi_ref):
    pid = pl.program_id(0)

    @pl.when(pid == 0)
    def _init():
        xn_ref[...] = _norm_x(xt_ref[...])
        runi_ref[...] = jnp.full((_B, 1), _BIG, jnp.int32)

    @pl.when(flag_ref[0] != 0)
    def _scan():
        _, mn = _norm_block(mt_ref, pid)
        sim = lax.dot_general(xn_ref[...], mn, (((0,), (0,)), ((), ())),
                              preferred_element_type=jnp.float32)
        lanes = lax.broadcasted_iota(jnp.int32, (1, _CH), 1)
        for r in range(_B // _RG):
            g = maxv_ref[pl.ds(r * _RG, _RG), :] - _EPS
            cand = jnp.full((_RG, _CH), jnp.int32(_BIG), jnp.int32)
            for k in range(_NCH):
                ck = lax.slice(sim, (r * _RG, k * _CH),
                               ((r + 1) * _RG, (k + 1) * _CH))
                cc = lanes + jnp.int32(k * _CH) + pid * _BLK
                cand = jnp.where(ck >= g, jnp.minimum(cand, cc), cand)
            bidx = jnp.min(cand, axis=1, keepdims=True)
            runi_ref[pl.ds(r * _RG, _RG), :] = jnp.minimum(
                runi_ref[pl.ds(r * _RG, _RG), :], bidx)

    @pl.when(pid == _NBLK - 1)
    def _fin():
        maxi_ref[...] = jnp.minimum(runi_ref[...], jnp.int32(_MEM - 1))


@functools.lru_cache(maxsize=1)
def _idx_call():
    return pl.pallas_call(
        _idx_body,
        grid_spec=pltpu.PrefetchScalarGridSpec(
            num_scalar_prefetch=1,
            grid=(_NBLK,),
            in_specs=[
                pl.BlockSpec((_DIM, _B), lambda i, f: (0, 0)),
                pl.BlockSpec((_DIM, _BLK), lambda i, f: (0, i * f[0])),
                pl.BlockSpec((_B, 1), lambda i, f: (0, 0)),
            ],
            out_specs=[
                pl.BlockSpec((_B, 1), lambda i, f: (0, 0)),
            ],
            scratch_shapes=[
                pltpu.VMEM((_DIM, _B), jnp.bfloat16),
                pltpu.VMEM((_B, 1), jnp.int32),
            ],
        ),
        out_shape=[
            jax.ShapeDtypeStruct((_B, 1), jnp.int32),
        ],
    )


_NC = 2
_NS = 16
_NW = _NC * _NS
_BW = _B // _NW


@functools.lru_cache(maxsize=1)
def _gather_call():
    mesh = plsc.VectorSubcoreMesh(core_axis_name="c", subcore_axis_name="s")

    @functools.partial(
        pl.kernel, mesh=mesh,
        out_type=jax.ShapeDtypeStruct((_B, _DIM), jnp.float32),
        scratch_types=[
            pltpu.VMEM((_BW,), jnp.int32),
            pltpu.VMEM((_BW, _DIM), jnp.float32),
            pltpu.VMEM((_BW,), jnp.float32),
            pltpu.SemaphoreType.DMA,
        ],
    )
    def k(table_hbm, idx_hbm, mval_hbm, out_hbm, idx_v, rows_v, mval_v, sem):
        wid = lax.axis_index("s") * _NC + lax.axis_index("c")
        base = wid * _BW
        pltpu.sync_copy(idx_hbm.at[pl.ds(base, _BW)], idx_v)
        pltpu.sync_copy(mval_hbm.at[pl.ds(base, _BW)], mval_v)
        copies = []
        for c2 in range(_BW // 16):
            iv = idx_v[pl.ds(c2 * 16, 16)]
            for l in range(16):
                i = c2 * 16 + l
                s = iv[l]
                copies.append(pltpu.async_copy(
                    table_hbm.at[pl.ds(s, 1)], rows_v.at[pl.ds(i, 1)], sem))
        for cp in copies:
            cp.wait()
        for c2 in range(_BW // 16):
            mv = mval_v[pl.ds(c2 * 16, 16)]
            maskvec = jnp.where(mv > _THR, jnp.float32(1.0), jnp.float32(0.0))
            for l in range(16):
                i = c2 * 16 + l
                m = maskvec[l]
                for c in range(_DIM // 16):
                    rows_v[i, pl.ds(c * 16, 16)] = rows_v[i, pl.ds(c * 16, 16)] * m
        pltpu.sync_copy(rows_v, out_hbm.at[pl.ds(base, _BW)])

    return k


def kernel(x, memory):
    xt = x.T
    mt = memory.T
    maxv, rowmaj = _max_call()(xt, mt)
    flag = jnp.any(maxv > _THR).astype(jnp.int32).reshape(1)
    (maxi,) = _idx_call()(flag, xt, mt, maxv)
    return _gather_call()(rowmaj, maxi.reshape(_B), maxv.reshape(_B))


# speedup vs baseline: 1.3796x; 1.3796x over previous
import functools

import jax
import jax.numpy as jnp
from jax import lax
from jax.experimental import pallas as pl
from jax.experimental.pallas import tpu as pltpu
from jax.experimental.pallas import tpu_sc as plsc

_MEM = 100000
_DIM = 64
_B = 1024
_BLK = 2048
_CH = 128
_NCH = _BLK // _CH
_RG = 128
_NBLK = -(-_MEM // _BLK)
_THR = 0.8
_BIG = 0x3FFFFFFF
_EPS = 7.6e-6


def _norm_x(xt):
    n = jnp.sqrt(jnp.sum(xt * xt, axis=0, keepdims=True))
    return (xt / jnp.maximum(n, 1e-12)).astype(jnp.bfloat16)


def _norm_block(mt_ref, pid):
    col = lax.broadcasted_iota(jnp.int32, (1, _BLK), 1)
    valid = (col + pid * _BLK) < _MEM
    mv = jnp.where(valid, mt_ref[...], 0.0)
    nsq = jnp.sum(mv * mv, axis=0, keepdims=True)
    rnorm = 1.0 / jnp.maximum(jnp.sqrt(nsq), 1e-12)
    return mv, (mv * rnorm).astype(jnp.bfloat16)


def _max_body(xt_ref, mt_ref, maxv_ref, rowmaj_ref, xn_ref, run_ref):
    pid = pl.program_id(0)

    @pl.when(pid == 0)
    def _init():
        xn_ref[...] = _norm_x(xt_ref[...])
        run_ref[...] = jnp.full((_B, _CH), -jnp.inf, jnp.float32)

    mv, mn = _norm_block(mt_ref, pid)
    rowmaj_ref[...] = mv.T

    sim = lax.dot_general(xn_ref[...], mn, (((0,), (0,)), ((), ())),
                          preferred_element_type=jnp.float32)
    for r in range(_B // _RG):
        acc = run_ref[pl.ds(r * _RG, _RG), :]
        for k in range(_NCH):
            acc = jnp.maximum(acc, lax.slice(
                sim, (r * _RG, k * _CH), ((r + 1) * _RG, (k + 1) * _CH)))
        run_ref[pl.ds(r * _RG, _RG), :] = acc

    @pl.when(pid == _NBLK - 1)
    def _fin():
        maxv_ref[...] = jnp.max(run_ref[...], axis=1, keepdims=True)


@functools.lru_cache(maxsize=1)
def _max_call():
    return pl.pallas_call(
        _max_body,
        grid=(_NBLK,),
        in_specs=[
            pl.BlockSpec((_DIM, _B), lambda i: (0, 0)),
            pl.BlockSpec((_DIM, _BLK), lambda i: (0, i)),
        ],
        out_specs=[
            pl.BlockSpec((_B, 1), lambda i: (0, 0)),
            pl.BlockSpec((_BLK, _DIM), lambda i: (i, 0)),
        ],
        out_shape=[
            jax.ShapeDtypeStruct((_B, 1), jnp.float32),
            jax.ShapeDtypeStruct((_NBLK * _BLK, _DIM), jnp.float32),
        ],
        scratch_shapes=[
            pltpu.VMEM((_DIM, _B), jnp.bfloat16),
            pltpu.VMEM((_B, _CH), jnp.float32),
        ],
    )


def _idx_body(flag_ref, xt_ref, mt_ref, maxv_ref, maxi_ref, xn_ref, runi_ref):
    pid = pl.program_id(0)

    @pl.when(pid == 0)
    def _init():
        xn_ref[...] = _norm_x(xt_ref[...])
        runi_ref[...] = jnp.full((_B, 1), _BIG, jnp.int32)

    @pl.when(flag_ref[0] != 0)
    def _scan():
        _, mn = _norm_block(mt_ref, pid)
        sim = lax.dot_general(xn_ref[...], mn, (((0,), (0,)), ((), ())),
                              preferred_element_type=jnp.float32)
        lanes = lax.broadcasted_iota(jnp.int32, (1, _CH), 1)
        for r in range(_B // _RG):
            g = maxv_ref[pl.ds(r * _RG, _RG), :] - _EPS
            cand = jnp.full((_RG, _CH), jnp.int32(_BIG), jnp.int32)
            for k in range(_NCH):
                ck = lax.slice(sim, (r * _RG, k * _CH),
                               ((r + 1) * _RG, (k + 1) * _CH))
                cc = lanes + jnp.int32(k * _CH) + pid * _BLK
                cand = jnp.where(ck >= g, jnp.minimum(cand, cc), cand)
            bidx = jnp.min(cand, axis=1, keepdims=True)
            runi_ref[pl.ds(r * _RG, _RG), :] = jnp.minimum(
                runi_ref[pl.ds(r * _RG, _RG), :], bidx)

    @pl.when(pid == _NBLK - 1)
    def _fin():
        qi = lax.broadcasted_iota(jnp.int32, (_B, 1), 0)
        runi = runi_ref[...]
        maxi_ref[...] = jnp.where(runi == jnp.int32(_BIG), qi,
                                  jnp.minimum(runi, jnp.int32(_MEM - 1)))


@functools.lru_cache(maxsize=1)
def _idx_call():
    return pl.pallas_call(
        _idx_body,
        grid_spec=pltpu.PrefetchScalarGridSpec(
            num_scalar_prefetch=1,
            grid=(_NBLK,),
            in_specs=[
                pl.BlockSpec((_DIM, _B), lambda i, f: (0, 0)),
                pl.BlockSpec((_DIM, _BLK), lambda i, f: (0, i * f[0])),
                pl.BlockSpec((_B, 1), lambda i, f: (0, 0)),
            ],
            out_specs=[
                pl.BlockSpec((_B, 1), lambda i, f: (0, 0)),
            ],
            scratch_shapes=[
                pltpu.VMEM((_DIM, _B), jnp.bfloat16),
                pltpu.VMEM((_B, 1), jnp.int32),
            ],
        ),
        out_shape=[
            jax.ShapeDtypeStruct((_B, 1), jnp.int32),
        ],
    )


_NC = 2
_NS = 16
_NW = _NC * _NS
_BW = _B // _NW


@functools.lru_cache(maxsize=1)
def _gather_call():
    mesh = plsc.VectorSubcoreMesh(core_axis_name="c", subcore_axis_name="s")

    @functools.partial(
        pl.kernel, mesh=mesh,
        out_type=jax.ShapeDtypeStruct((_B, _DIM), jnp.float32),
        scratch_types=[
            pltpu.VMEM((_BW,), jnp.int32),
            pltpu.VMEM((_BW, _DIM), jnp.float32),
            pltpu.VMEM((_BW,), jnp.float32),
            pltpu.SemaphoreType.DMA,
        ],
    )
    def k(table_hbm, idx_hbm, mval_hbm, out_hbm, idx_v, rows_v, mval_v, sem):
        wid = lax.axis_index("s") * _NC + lax.axis_index("c")
        base = wid * _BW
        pltpu.sync_copy(idx_hbm.at[pl.ds(base, _BW)], idx_v)
        pltpu.sync_copy(mval_hbm.at[pl.ds(base, _BW)], mval_v)
        copies = []
        for c2 in range(_BW // 16):
            iv = idx_v[pl.ds(c2 * 16, 16)]
            for l in range(16):
                i = c2 * 16 + l
                s = iv[l]
                copies.append(pltpu.async_copy(
                    table_hbm.at[pl.ds(s, 1)], rows_v.at[pl.ds(i, 1)], sem))
        for cp in copies:
            cp.wait()
        for c2 in range(_BW // 16):
            mv = mval_v[pl.ds(c2 * 16, 16)]
            maskvec = jnp.where(mv > _THR, jnp.float32(1.0), jnp.float32(0.0))
            for l in range(16):
                i = c2 * 16 + l
                m = maskvec[l]
                for c in range(_DIM // 16):
                    rows_v[i, pl.ds(c * 16, 16)] = rows_v[i, pl.ds(c * 16, 16)] * m
        pltpu.sync_copy(rows_v, out_hbm.at[pl.ds(base, _BW)])

    return k


def kernel(x, memory):
    xt = x.T
    mt = memory.T
    maxv, rowmaj = _max_call()(xt, mt)
    flag = jnp.any(maxv > _THR).astype(jnp.int32).reshape(1)
    (maxi,) = _idx_call()(flag, xt, mt, maxv)
    return _gather_call()(rowmaj, maxi.reshape(_B), maxv.reshape(_B))


# speedup vs baseline: 1.4948x; 1.0835x over previous
import functools

import jax
import jax.numpy as jnp
from jax import lax
from jax.experimental import pallas as pl
from jax.experimental.pallas import tpu as pltpu
from jax.experimental.pallas import tpu_sc as plsc

_MEM = 100000
_DIM = 64
_B = 1024
_BLK = 4096
_CH = 128
_NCH = _BLK // _CH
_RG = 128
_NBLK = -(-_MEM // _BLK)
_THR = 0.8
_BIG = 0x3FFFFFFF
_EPS = 7.6e-6


def _norm_x(xt):
    n = jnp.sqrt(jnp.sum(xt * xt, axis=0, keepdims=True))
    return (xt / jnp.maximum(n, 1e-12)).astype(jnp.bfloat16)


def _norm_block(mt_ref, pid):
    col = lax.broadcasted_iota(jnp.int32, (1, _BLK), 1)
    valid = (col + pid * _BLK) < _MEM
    mv = jnp.where(valid, mt_ref[...], 0.0)
    nsq = jnp.sum(mv * mv, axis=0, keepdims=True)
    rnorm = 1.0 / jnp.maximum(jnp.sqrt(nsq), 1e-12)
    return mv, (mv * rnorm).astype(jnp.bfloat16)


def _max_body(xt_ref, mt_ref, maxv_ref, rowmaj_ref, xn_ref, run_ref):
    pid = pl.program_id(0)

    @pl.when(pid == 0)
    def _init():
        xn_ref[...] = _norm_x(xt_ref[...])
        run_ref[...] = jnp.full((_B, _CH), -jnp.inf, jnp.float32)

    mv, mn = _norm_block(mt_ref, pid)
    rowmaj_ref[...] = mv.T

    sim = lax.dot_general(xn_ref[...], mn, (((0,), (0,)), ((), ())),
                          preferred_element_type=jnp.float32)
    for r in range(_B // _RG):
        acc = run_ref[pl.ds(r * _RG, _RG), :]
        for k in range(_NCH):
            acc = jnp.maximum(acc, lax.slice(
                sim, (r * _RG, k * _CH), ((r + 1) * _RG, (k + 1) * _CH)))
        run_ref[pl.ds(r * _RG, _RG), :] = acc

    @pl.when(pid == _NBLK - 1)
    def _fin():
        maxv_ref[...] = jnp.max(run_ref[...], axis=1, keepdims=True)


@functools.lru_cache(maxsize=1)
def _max_call():
    return pl.pallas_call(
        _max_body,
        grid=(_NBLK,),
        in_specs=[
            pl.BlockSpec((_DIM, _B), lambda i: (0, 0)),
            pl.BlockSpec((_DIM, _BLK), lambda i: (0, i)),
        ],
        out_specs=[
            pl.BlockSpec((_B, 1), lambda i: (0, 0)),
            pl.BlockSpec((_BLK, _DIM), lambda i: (i, 0)),
        ],
        out_shape=[
            jax.ShapeDtypeStruct((_B, 1), jnp.float32),
            jax.ShapeDtypeStruct((_NBLK * _BLK, _DIM), jnp.float32),
        ],
        scratch_shapes=[
            pltpu.VMEM((_DIM, _B), jnp.bfloat16),
            pltpu.VMEM((_B, _CH), jnp.float32),
        ],
    )


def _idx_body(flag_ref, xt_ref, mt_ref, maxv_ref, maxi_ref, xn_ref, runi_ref):
    pid = pl.program_id(0)

    @pl.when(pid == 0)
    def _init():
        xn_ref[...] = _norm_x(xt_ref[...])
        runi_ref[...] = jnp.full((_B, 1), _BIG, jnp.int32)

    @pl.when(flag_ref[0] != 0)
    def _scan():
        _, mn = _norm_block(mt_ref, pid)
        sim = lax.dot_general(xn_ref[...], mn, (((0,), (0,)), ((), ())),
                              preferred_element_type=jnp.float32)
        lanes = lax.broadcasted_iota(jnp.int32, (1, _CH), 1)
        for r in range(_B // _RG):
            g = maxv_ref[pl.ds(r * _RG, _RG), :] - _EPS
            cand = jnp.full((_RG, _CH), jnp.int32(_BIG), jnp.int32)
            for k in range(_NCH):
                ck = lax.slice(sim, (r * _RG, k * _CH),
                               ((r + 1) * _RG, (k + 1) * _CH))
                cc = lanes + jnp.int32(k * _CH) + pid * _BLK
                cand = jnp.where(ck >= g, jnp.minimum(cand, cc), cand)
            bidx = jnp.min(cand, axis=1, keepdims=True)
            runi_ref[pl.ds(r * _RG, _RG), :] = jnp.minimum(
                runi_ref[pl.ds(r * _RG, _RG), :], bidx)

    @pl.when(pid == _NBLK - 1)
    def _fin():
        qi = lax.broadcasted_iota(jnp.int32, (_B, 1), 0)
        runi = runi_ref[...]
        maxi_ref[...] = jnp.where(runi == jnp.int32(_BIG), qi,
                                  jnp.minimum(runi, jnp.int32(_MEM - 1)))


@functools.lru_cache(maxsize=1)
def _idx_call():
    return pl.pallas_call(
        _idx_body,
        grid_spec=pltpu.PrefetchScalarGridSpec(
            num_scalar_prefetch=1,
            grid=(_NBLK,),
            in_specs=[
                pl.BlockSpec((_DIM, _B), lambda i, f: (0, 0)),
                pl.BlockSpec((_DIM, _BLK), lambda i, f: (0, i * f[0])),
                pl.BlockSpec((_B, 1), lambda i, f: (0, 0)),
            ],
            out_specs=[
                pl.BlockSpec((_B, 1), lambda i, f: (0, 0)),
            ],
            scratch_shapes=[
                pltpu.VMEM((_DIM, _B), jnp.bfloat16),
                pltpu.VMEM((_B, 1), jnp.int32),
            ],
        ),
        out_shape=[
            jax.ShapeDtypeStruct((_B, 1), jnp.int32),
        ],
    )


_NC = 2
_NS = 16
_NW = _NC * _NS
_BW = _B // _NW


@functools.lru_cache(maxsize=1)
def _gather_call():
    mesh = plsc.VectorSubcoreMesh(core_axis_name="c", subcore_axis_name="s")

    @functools.partial(
        pl.kernel, mesh=mesh,
        out_type=jax.ShapeDtypeStruct((_B, _DIM), jnp.float32),
        scratch_types=[
            pltpu.VMEM((_BW,), jnp.int32),
            pltpu.VMEM((_BW, _DIM), jnp.float32),
            pltpu.VMEM((_BW,), jnp.float32),
            pltpu.SemaphoreType.DMA,
        ],
    )
    def k(table_hbm, idx_hbm, mval_hbm, out_hbm, idx_v, rows_v, mval_v, sem):
        wid = lax.axis_index("s") * _NC + lax.axis_index("c")
        base = wid * _BW
        pltpu.sync_copy(idx_hbm.at[pl.ds(base, _BW)], idx_v)
        pltpu.sync_copy(mval_hbm.at[pl.ds(base, _BW)], mval_v)
        copies = []
        for c2 in range(_BW // 16):
            iv = idx_v[pl.ds(c2 * 16, 16)]
            for l in range(16):
                i = c2 * 16 + l
                s = iv[l]
                copies.append(pltpu.async_copy(
                    table_hbm.at[pl.ds(s, 1)], rows_v.at[pl.ds(i, 1)], sem))
        for cp in copies:
            cp.wait()
        for c2 in range(_BW // 16):
            mv = mval_v[pl.ds(c2 * 16, 16)]
            maskvec = jnp.where(mv > _THR, jnp.float32(1.0), jnp.float32(0.0))
            for l in range(16):
                i = c2 * 16 + l
                m = maskvec[l]
                for c in range(_DIM // 16):
                    rows_v[i, pl.ds(c * 16, 16)] = rows_v[i, pl.ds(c * 16, 16)] * m
        pltpu.sync_copy(rows_v, out_hbm.at[pl.ds(base, _BW)])

    return k


def kernel(x, memory):
    xt = x.T
    mt = memory.T
    maxv, rowmaj = _max_call()(xt, mt)
    flag = jnp.any(maxv > _THR).astype(jnp.int32).reshape(1)
    (maxi,) = _idx_call()(flag, xt, mt, maxv)
    return _gather_call()(rowmaj, maxi.reshape(_B), maxv.reshape(_B))


# speedup vs baseline: 1.4959x; 1.0007x over previous
import functools

import jax
import jax.numpy as jnp
from jax import lax
from jax.experimental import pallas as pl
from jax.experimental.pallas import tpu as pltpu
from jax.experimental.pallas import tpu_sc as plsc

_MEM = 100000
_DIM = 64
_B = 1024
_BLK = 8192
_CH = 128
_NCH = _BLK // _CH
_RG = 128
_NBLK = -(-_MEM // _BLK)
_THR = 0.8
_BIG = 0x3FFFFFFF
_EPS = 7.6e-6


def _norm_x(xt):
    n = jnp.sqrt(jnp.sum(xt * xt, axis=0, keepdims=True))
    return (xt / jnp.maximum(n, 1e-12)).astype(jnp.bfloat16)


def _norm_block(mt_ref, pid):
    col = lax.broadcasted_iota(jnp.int32, (1, _BLK), 1)
    valid = (col + pid * _BLK) < _MEM
    mv = jnp.where(valid, mt_ref[...], 0.0)
    nsq = jnp.sum(mv * mv, axis=0, keepdims=True)
    rnorm = 1.0 / jnp.maximum(jnp.sqrt(nsq), 1e-12)
    return mv, (mv * rnorm).astype(jnp.bfloat16)


def _max_body(xt_ref, mt_ref, maxv_ref, rowmaj_ref, xn_ref, run_ref):
    pid = pl.program_id(0)

    @pl.when(pid == 0)
    def _init():
        xn_ref[...] = _norm_x(xt_ref[...])
        run_ref[...] = jnp.full((_B, _CH), -jnp.inf, jnp.float32)

    mv, mn = _norm_block(mt_ref, pid)
    rowmaj_ref[...] = mv.T

    sim = lax.dot_general(xn_ref[...], mn, (((0,), (0,)), ((), ())),
                          preferred_element_type=jnp.float32)
    for r in range(_B // _RG):
        acc = run_ref[pl.ds(r * _RG, _RG), :]
        for k in range(_NCH):
            acc = jnp.maximum(acc, lax.slice(
                sim, (r * _RG, k * _CH), ((r + 1) * _RG, (k + 1) * _CH)))
        run_ref[pl.ds(r * _RG, _RG), :] = acc

    @pl.when(pid == _NBLK - 1)
    def _fin():
        maxv_ref[...] = jnp.max(run_ref[...], axis=1, keepdims=True)


@functools.lru_cache(maxsize=1)
def _max_call():
    return pl.pallas_call(
        _max_body,
        grid=(_NBLK,),
        in_specs=[
            pl.BlockSpec((_DIM, _B), lambda i: (0, 0)),
            pl.BlockSpec((_DIM, _BLK), lambda i: (0, i)),
        ],
        out_specs=[
            pl.BlockSpec((_B, 1), lambda i: (0, 0)),
            pl.BlockSpec((_BLK, _DIM), lambda i: (i, 0)),
        ],
        out_shape=[
            jax.ShapeDtypeStruct((_B, 1), jnp.float32),
            jax.ShapeDtypeStruct((_NBLK * _BLK, _DIM), jnp.float32),
        ],
        scratch_shapes=[
            pltpu.VMEM((_DIM, _B), jnp.bfloat16),
            pltpu.VMEM((_B, _CH), jnp.float32),
        ],
    )


def _idx_body(flag_ref, xt_ref, mt_ref, maxv_ref, maxi_ref, xn_ref, runi_ref):
    pid = pl.program_id(0)

    @pl.when(pid == 0)
    def _init():
        xn_ref[...] = _norm_x(xt_ref[...])
        runi_ref[...] = jnp.full((_B, 1), _BIG, jnp.int32)

    @pl.when(flag_ref[0] != 0)
    def _scan():
        _, mn = _norm_block(mt_ref, pid)
        sim = lax.dot_general(xn_ref[...], mn, (((0,), (0,)), ((), ())),
                              preferred_element_type=jnp.float32)
        lanes = lax.broadcasted_iota(jnp.int32, (1, _CH), 1)
        for r in range(_B // _RG):
            g = maxv_ref[pl.ds(r * _RG, _RG), :] - _EPS
            cand = jnp.full((_RG, _CH), jnp.int32(_BIG), jnp.int32)
            for k in range(_NCH):
                ck = lax.slice(sim, (r * _RG, k * _CH),
                               ((r + 1) * _RG, (k + 1) * _CH))
                cc = lanes + jnp.int32(k * _CH) + pid * _BLK
                cand = jnp.where(ck >= g, jnp.minimum(cand, cc), cand)
            bidx = jnp.min(cand, axis=1, keepdims=True)
            runi_ref[pl.ds(r * _RG, _RG), :] = jnp.minimum(
                runi_ref[pl.ds(r * _RG, _RG), :], bidx)

    @pl.when(pid == _NBLK - 1)
    def _fin():
        qi = lax.broadcasted_iota(jnp.int32, (_B, 1), 0)
        runi = runi_ref[...]
        maxi_ref[...] = jnp.where(runi == jnp.int32(_BIG), qi,
                                  jnp.minimum(runi, jnp.int32(_MEM - 1)))


@functools.lru_cache(maxsize=1)
def _idx_call():
    return pl.pallas_call(
        _idx_body,
        grid_spec=pltpu.PrefetchScalarGridSpec(
            num_scalar_prefetch=1,
            grid=(_NBLK,),
            in_specs=[
                pl.BlockSpec((_DIM, _B), lambda i, f: (0, 0)),
                pl.BlockSpec((_DIM, _BLK), lambda i, f: (0, i * f[0])),
                pl.BlockSpec((_B, 1), lambda i, f: (0, 0)),
            ],
            out_specs=[
                pl.BlockSpec((_B, 1), lambda i, f: (0, 0)),
            ],
            scratch_shapes=[
                pltpu.VMEM((_DIM, _B), jnp.bfloat16),
                pltpu.VMEM((_B, 1), jnp.int32),
            ],
        ),
        out_shape=[
            jax.ShapeDtypeStruct((_B, 1), jnp.int32),
        ],
    )


_NC = 2
_NS = 16
_NW = _NC * _NS
_BW = _B // _NW


@functools.lru_cache(maxsize=1)
def _gather_call():
    mesh = plsc.VectorSubcoreMesh(core_axis_name="c", subcore_axis_name="s")

    @functools.partial(
        pl.kernel, mesh=mesh,
        out_type=jax.ShapeDtypeStruct((_B, _DIM), jnp.float32),
        scratch_types=[
            pltpu.VMEM((_BW,), jnp.int32),
            pltpu.VMEM((_BW, _DIM), jnp.float32),
            pltpu.VMEM((_BW,), jnp.float32),
            pltpu.SemaphoreType.DMA,
        ],
    )
    def k(table_hbm, idx_hbm, mval_hbm, out_hbm, idx_v, rows_v, mval_v, sem):
        wid = lax.axis_index("s") * _NC + lax.axis_index("c")
        base = wid * _BW
        pltpu.sync_copy(idx_hbm.at[pl.ds(base, _BW)], idx_v)
        pltpu.sync_copy(mval_hbm.at[pl.ds(base, _BW)], mval_v)
        copies = []
        for c2 in range(_BW // 16):
            iv = idx_v[pl.ds(c2 * 16, 16)]
            for l in range(16):
                i = c2 * 16 + l
                s = iv[l]
                copies.append(pltpu.async_copy(
                    table_hbm.at[pl.ds(s, 1)], rows_v.at[pl.ds(i, 1)], sem))
        for cp in copies:
            cp.wait()
        for c2 in range(_BW // 16):
            mv = mval_v[pl.ds(c2 * 16, 16)]
            maskvec = jnp.where(mv > _THR, jnp.float32(1.0), jnp.float32(0.0))
            for l in range(16):
                i = c2 * 16 + l
                m = maskvec[l]
                for c in range(_DIM // 16):
                    rows_v[i, pl.ds(c * 16, 16)] = rows_v[i, pl.ds(c * 16, 16)] * m
        pltpu.sync_copy(rows_v, out_hbm.at[pl.ds(base, _BW)])

    return k


def kernel(x, memory):
    xt = x.T
    mt = memory.T
    maxv, rowmaj = _max_call()(xt, mt)
    flag = jnp.any(maxv > _THR).astype(jnp.int32).reshape(1)
    (maxi,) = _idx_call()(flag, xt, mt, maxv)
    return _gather_call()(rowmaj, maxi.reshape(_B), maxv.reshape(_B))
